# baseline (speedup 1.0000x reference)
import jax
import jax.numpy as jnp
from jax import lax
from jax.experimental import pallas as pl
from jax.experimental.pallas import tpu as pltpu
from jax.experimental.pallas import tpu_sc as plsc

B = 16384
L = 21
S = 20
D = 32
HIDDEN = 256
NP = 1000000

NC, NS = 2, 16
NW = NC * NS
BPW = B // NW
CH = 16
NCH = BPW // CH


def _gather_sum_body(lineup_hbm, table_hbm, out_hbm, lin_v, idx_v, rv_v,
                     rows_v, outc_v, sem):
    wid = lax.axis_index("s") * NC + lax.axis_index("c")
    base = wid * BPW

    pltpu.sync_copy(lineup_hbm.at[pl.ds(base * L, BPW * L)], lin_v)

    lane = lax.iota(jnp.int32, 16)
    lane21 = lane * L

    def chunk_body(c, carry):
        for j in range(S):
            p = plsc.load_gather(lin_v, [lane21 + (c * (CH * L) + j)])
            idx_v[j, :] = lax.shift_right_logical(p, 2)
            rv_v[j, :] = lax.bitwise_and(p, 3) * D

        descs = []
        for j in range(S):
            d = pltpu.async_copy(
                table_hbm.at[idx_v.at[j]], rows_v.at[pl.ds(j * CH, CH)], sem
            )
            descs.append(d)
        for d in descs:
            d.wait()

        acc = [jnp.zeros((16,), jnp.float32) for _ in range(D)]
        for j in range(S):
            tvec = lane + (j * CH)
            rv = rv_v[j, :]
            for d in range(D):
                acc[d] = acc[d] + plsc.load_gather(rows_v, [tvec, rv + d])

        for d in range(D):
            plsc.store_scatter(outc_v, [lane, jnp.full((16,), d, jnp.int32)],
                               acc[d])
        pltpu.sync_copy(outc_v, out_hbm.at[pl.ds(base + c * CH, CH)])
        return carry

    lax.fori_loop(0, NCH, chunk_body, 0)


def _gather_sum(lineup_flat, table3):
    mesh = plsc.VectorSubcoreMesh(
        core_axis_name="c", subcore_axis_name="s", num_cores=NC, num_subcores=NS
    )
    return pl.kernel(
        _gather_sum_body,
        out_type=jax.ShapeDtypeStruct((B, D), jnp.float32),
        mesh=mesh,
        scratch_types=[
            pltpu.VMEM((BPW * L,), jnp.int32),
            pltpu.VMEM((S, 16), jnp.int32),
            pltpu.VMEM((S, 16), jnp.int32),
            pltpu.VMEM((S * CH, 4 * D), jnp.float32),
            pltpu.VMEM((CH, D), jnp.float32),
            pltpu.SemaphoreType.DMA,
        ],
        compiler_params=pltpu.CompilerParams(needs_layout_passes=False),
    )(lineup_flat, table3)


def _mlp_body(x_ref, lineup_ref, w1t_ref, b1_ref, w2_ref, b2_ref, o_ref):
    x = x_ref[...]
    flag = lineup_ref[:, S:].astype(jnp.float32)
    x33 = jnp.concatenate([x, flag], axis=1)
    h = jnp.dot(x33, w1t_ref[...], preferred_element_type=jnp.float32)
    h = jnp.maximum(h + b1_ref[...], 0.0)
    o = jnp.dot(h, w2_ref[...], preferred_element_type=jnp.float32)
    o_ref[...] = o + b2_ref[0, 0]


def _mlp(summed, lineup, w1t, b1_2d, w2t, b2_2d):
    BT = 2048
    grid = (B // BT,)
    return pl.pallas_call(
        _mlp_body,
        grid=grid,
        in_specs=[
            pl.BlockSpec((BT, D), lambda i: (i, 0)),
            pl.BlockSpec((BT, L), lambda i: (i, 0)),
            pl.BlockSpec((D + 1, HIDDEN), lambda i: (0, 0)),
            pl.BlockSpec((1, HIDDEN), lambda i: (0, 0)),
            pl.BlockSpec((HIDDEN, 1), lambda i: (0, 0)),
            pl.BlockSpec((1, 1), lambda i: (0, 0)),
        ],
        out_specs=pl.BlockSpec((BT, 1), lambda i: (i, 0)),
        out_shape=jax.ShapeDtypeStruct((B, 1), jnp.float32),
    )(summed, lineup, w1t, b1_2d, w2t, b2_2d)


def kernel(lineup, table, W1, b1, W2, b2):
    table4 = table.reshape(NP // 4, 4 * D)
    summed = _gather_sum(lineup.reshape(-1), table4)
    return _mlp(summed, lineup, W1.T, b1.reshape(1, HIDDEN), W2.T,
                b2.reshape(1, 1))


# speedup vs baseline: 1.1841x; 1.1841x over previous
import jax
import jax.numpy as jnp
from jax import lax
from jax.experimental import pallas as pl
from jax.experimental.pallas import tpu as pltpu
from jax.experimental.pallas import tpu_sc as plsc

B = 16384
L = 21
S = 20
D = 32
HIDDEN = 256
NP = 1000000
WIDE = 128

NC, NS = 2, 16
NW = NC * NS
BPW = B // NW
CHUNK = 128
NCHUNK = BPW // CHUNK

RBLK = 8192


def _repack_body(tt_ref, out_ref):
    blk = tt_ref[...]
    t = jnp.transpose(blk, (1, 0))
    out_ref[...] = jnp.concatenate([t, t, t, t], axis=1)


def _repack(tableT):
    grid = ((NP + RBLK - 1) // RBLK,)
    return pl.pallas_call(
        _repack_body,
        grid=grid,
        in_specs=[pl.BlockSpec((D, RBLK), lambda i: (0, i))],
        out_specs=pl.BlockSpec((RBLK, WIDE), lambda i: (i, 0)),
        out_shape=jax.ShapeDtypeStruct((NP, WIDE), jnp.float32),
    )(tableT)


def _gather_sum_body(lineup_hbm, table_hbm, out_hbm, lin_v, idx_v, acc_v, sem):
    wid = lax.axis_index("s") * NC + lax.axis_index("c")
    base = wid * BPW

    pltpu.sync_copy(lineup_hbm.at[pl.ds(base * L, BPW * L)], lin_v)

    zero = jnp.zeros((16,), jnp.float32)

    def zero_body(i, carry):
        for u in range(4):
            r = i * 4 + u
            acc_v[r, pl.ds(0, 16)] = zero
            acc_v[r, pl.ds(16, 16)] = zero
        return carry

    lax.fori_loop(0, BPW // 4, zero_body, 0)

    lane = lax.iota(jnp.int32, 16)
    lane21 = lane * L

    def build_body(t, carry):
        c = t // S
        j = t % S
        rowbase = c * CHUNK
        for g in range(CHUNK // 16):
            flat = lane21 + ((rowbase + g * 16) * L + j)
            idx_v[t, pl.ds(g * 16, 16)] = plsc.load_gather(lin_v, [flat])
        return carry

    lax.fori_loop(0, NCHUNK * S, build_body, 0)

    def chunk_body(c, carry):
        dst = acc_v.at[pl.ds(c * CHUNK, CHUNK)]
        descs = []
        for j in range(S):
            d = pltpu.async_copy(
                table_hbm.at[idx_v.at[c * S + j]], dst, sem, add=True
            )
            descs.append(d)
        for d in descs:
            d.wait()
        return carry

    lax.fori_loop(0, NCHUNK, chunk_body, 0)

    pltpu.sync_copy(acc_v, out_hbm.at[pl.ds(base, BPW)])


def _gather_sum(lineup_flat, table_wide):
    mesh = plsc.VectorSubcoreMesh(
        core_axis_name="c", subcore_axis_name="s", num_cores=NC, num_subcores=NS
    )
    return pl.kernel(
        _gather_sum_body,
        out_type=jax.ShapeDtypeStruct((B, WIDE), jnp.float32),
        mesh=mesh,
        scratch_types=[
            pltpu.VMEM((BPW * L,), jnp.int32),
            pltpu.VMEM((NCHUNK * S, CHUNK), jnp.int32),
            pltpu.VMEM((BPW, WIDE), jnp.float32),
            pltpu.SemaphoreType.DMA,
        ],
        compiler_params=pltpu.CompilerParams(needs_layout_passes=False),
    )(lineup_flat, table_wide)


def _mlp_body(x_ref, lineup_ref, w1t_ref, b1_ref, w2_ref, b2_ref, o_ref):
    x = x_ref[:, :D]
    flag = lineup_ref[:, S:].astype(jnp.float32)
    x33 = jnp.concatenate([x, flag], axis=1)
    h = jnp.dot(x33, w1t_ref[...], preferred_element_type=jnp.float32)
    h = jnp.maximum(h + b1_ref[...], 0.0)
    o = jnp.dot(h, w2_ref[...], preferred_element_type=jnp.float32)
    o_ref[...] = o + b2_ref[0, 0]


def _mlp(summed, lineup, w1t, b1_2d, w2t, b2_2d):
    BT = 2048
    grid = (B // BT,)
    return pl.pallas_call(
        _mlp_body,
        grid=grid,
        in_specs=[
            pl.BlockSpec((BT, WIDE), lambda i: (i, 0)),
            pl.BlockSpec((BT, L), lambda i: (i, 0)),
            pl.BlockSpec((D + 1, HIDDEN), lambda i: (0, 0)),
            pl.BlockSpec((1, HIDDEN), lambda i: (0, 0)),
            pl.BlockSpec((HIDDEN, 1), lambda i: (0, 0)),
            pl.BlockSpec((1, 1), lambda i: (0, 0)),
        ],
        out_specs=pl.BlockSpec((BT, 1), lambda i: (i, 0)),
        out_shape=jax.ShapeDtypeStruct((B, 1), jnp.float32),
    )(summed, lineup, w1t, b1_2d, w2t, b2_2d)


def kernel(lineup, table, W1, b1, W2, b2):
    table_wide = _repack(table.T)
    summed = _gather_sum(lineup.reshape(-1), table_wide)
    return _mlp(summed, lineup, W1.T, b1.reshape(1, HIDDEN), W2.T,
                b2.reshape(1, 1))


# speedup vs baseline: 1.4981x; 1.2652x over previous
import jax
import jax.numpy as jnp
from jax import lax
from jax.experimental import pallas as pl
from jax.experimental.pallas import tpu as pltpu
from jax.experimental.pallas import tpu_sc as plsc

B = 16384
L = 21
S = 20
D = 32
HIDDEN = 256
NP = 1000000
WIDE = 128

RBLK = 8192
QB = RBLK // 4
NGRID = (NP + RBLK - 1) // RBLK
NLINES = NGRID * QB

NC, NS = 2, 16
NW = NC * NS
BPW = B // NW
CH = 16
NCH = BPW // CH


def _repack_body(tt_ref, out_ref):
    qs = []
    for q in range(4):
        blk = tt_ref[:, pl.ds(q * QB, QB)]
        qs.append(jnp.transpose(blk, (1, 0)))
    out_ref[...] = jnp.concatenate(qs, axis=1)


def _repack(tableT):
    return pl.pallas_call(
        _repack_body,
        grid=(NGRID,),
        in_specs=[pl.BlockSpec((D, RBLK), lambda i: (0, i))],
        out_specs=pl.BlockSpec((QB, WIDE), lambda i: (i, 0)),
        out_shape=jax.ShapeDtypeStruct((NLINES, WIDE), jnp.float32),
    )(tableT)


def _gather_sum_body(lineup_hbm, table_hbm, out_hbm, lin_v, idx0_v, idx1_v,
                     rv0_v, rv1_v, rows0_v, rows1_v, outc_v, sem0, sem1):
    wid = lax.axis_index("s") * NC + lax.axis_index("c")
    base = wid * BPW

    pltpu.sync_copy(lineup_hbm.at[pl.ds(base * L, BPW * L)], lin_v)

    lane = lax.iota(jnp.int32, 16)
    lane21 = lane * L
    idx_b = [idx0_v, idx1_v]
    rv_b = [rv0_v, rv1_v]
    rows_b = [rows0_v, rows1_v]
    sem_b = [sem0, sem1]

    def stage_chunk(c, buf):
        for j in range(S):
            p = plsc.load_gather(lin_v, [lane21 + (c * (CH * L) + j)])
            line = lax.bitwise_or(
                lax.shift_left(lax.shift_right_logical(p, 13), 11),
                lax.bitwise_and(p, 2047),
            )
            off = lax.shift_left(
                lax.bitwise_and(lax.shift_right_logical(p, 11), 3), 5)
            idx_b[buf][j, :] = line
            rv_b[buf][j, :] = off
        for j in range(S):
            pltpu.async_copy(
                table_hbm.at[idx_b[buf].at[j]],
                rows_b[buf].at[pl.ds(j * CH, CH)],
                sem_b[buf],
            )

    def drain_extract(c, buf):
        pltpu.make_async_copy(
            table_hbm.at[pl.ds(0, S * CH)], rows_b[buf], sem_b[buf]
        ).wait()
        tr = [(lane + (j * CH), rv_b[buf][j, :]) for j in range(S)]
        for d in range(D):
            acc = jnp.zeros((16,), jnp.float32)
            for tvec, rv in tr:
                acc = acc + plsc.load_gather(rows_b[buf], [tvec, rv + d])
            plsc.store_scatter(outc_v, [lane, jnp.full((16,), d, jnp.int32)],
                               acc)
        pltpu.sync_copy(outc_v, out_hbm.at[pl.ds(base + c * CH, CH)])

    stage_chunk(0, 0)

    def pair_body(g, carry):
        c0 = g * 2
        stage_chunk(c0 + 1, 1)
        drain_extract(c0, 0)

        @pl.when(g < (NCH // 2) - 1)
        def _():
            stage_chunk(c0 + 2, 0)

        drain_extract(c0 + 1, 1)
        return carry

    lax.fori_loop(0, NCH // 2, pair_body, 0)


def _gather_sum(lineup_flat, table_lines):
    mesh = plsc.VectorSubcoreMesh(
        core_axis_name="c", subcore_axis_name="s", num_cores=NC, num_subcores=NS
    )
    return pl.kernel(
        _gather_sum_body,
        out_type=jax.ShapeDtypeStruct((B, D), jnp.float32),
        mesh=mesh,
        scratch_types=[
            pltpu.VMEM((BPW * L,), jnp.int32),
            pltpu.VMEM((S, 16), jnp.int32),
            pltpu.VMEM((S, 16), jnp.int32),
            pltpu.VMEM((S, 16), jnp.int32),
            pltpu.VMEM((S, 16), jnp.int32),
            pltpu.VMEM((S * CH, WIDE), jnp.float32),
            pltpu.VMEM((S * CH, WIDE), jnp.float32),
            pltpu.VMEM((CH, D), jnp.float32),
            pltpu.SemaphoreType.DMA,
            pltpu.SemaphoreType.DMA,
        ],
        compiler_params=pltpu.CompilerParams(needs_layout_passes=False),
    )(lineup_flat, table_lines)


def _mlp_body(x_ref, lineup_ref, w1t_ref, b1_ref, w2_ref, b2_ref, o_ref):
    x = x_ref[...]
    flag = lineup_ref[:, S:].astype(jnp.float32)
    x33 = jnp.concatenate([x, flag], axis=1)
    h = jnp.dot(x33, w1t_ref[...], preferred_element_type=jnp.float32)
    h = jnp.maximum(h + b1_ref[...], 0.0)
    o = jnp.dot(h, w2_ref[...], preferred_element_type=jnp.float32)
    o_ref[...] = o + b2_ref[0, 0]


def _mlp(summed, lineup, w1t, b1_2d, w2t, b2_2d):
    BT = 2048
    grid = (B // BT,)
    return pl.pallas_call(
        _mlp_body,
        grid=grid,
        in_specs=[
            pl.BlockSpec((BT, D), lambda i: (i, 0)),
            pl.BlockSpec((BT, L), lambda i: (i, 0)),
            pl.BlockSpec((D + 1, HIDDEN), lambda i: (0, 0)),
            pl.BlockSpec((1, HIDDEN), lambda i: (0, 0)),
            pl.BlockSpec((HIDDEN, 1), lambda i: (0, 0)),
            pl.BlockSpec((1, 1), lambda i: (0, 0)),
        ],
        out_specs=pl.BlockSpec((BT, 1), lambda i: (i, 0)),
        out_shape=jax.ShapeDtypeStruct((B, 1), jnp.float32),
    )(summed, lineup, w1t, b1_2d, w2t, b2_2d)


def kernel(lineup, table, W1, b1, W2, b2):
    table_lines = _repack(table.T)
    summed = _gather_sum(lineup.reshape(-1), table_lines)
    return _mlp(summed, lineup, W1.T, b1.reshape(1, HIDDEN), W2.T,
                b2.reshape(1, 1))


# speedup vs baseline: 1.5070x; 1.0059x over previous
import jax
import jax.numpy as jnp
from jax import lax
from jax.experimental import pallas as pl
from jax.experimental.pallas import tpu as pltpu
from jax.experimental.pallas import tpu_sc as plsc

B = 16384
L = 21
S = 20
D = 32
HIDDEN = 256
NP = 1000000
WIDE = 128

RBLK = 8192
QB = RBLK // 4
NGRID = (NP + RBLK - 1) // RBLK
NLINES = NGRID * QB

NC, NS = 2, 16
NW = NC * NS
BPW = B // NW
CH = 16
NCH = BPW // CH


def _repack_body(tt_ref, out_ref):
    qs = []
    for q in range(4):
        blk = tt_ref[:, pl.ds(q * QB, QB)]
        qs.append(jnp.transpose(blk, (1, 0)))
    out_ref[...] = jnp.concatenate(qs, axis=1)


def _repack(tableT):
    return pl.pallas_call(
        _repack_body,
        grid=(NGRID,),
        in_specs=[pl.BlockSpec((D, RBLK), lambda i: (0, i))],
        out_specs=pl.BlockSpec((QB, WIDE), lambda i: (i, 0)),
        out_shape=jax.ShapeDtypeStruct((NLINES, WIDE), jnp.float32),
    )(tableT)


def _gather_sum_body(lineup_hbm, table_hbm, out_hbm, lin_v, idx0_v, idx1_v,
                     rv0_v, rv1_v, rows0_v, rows1_v, outc_v, sem0, sem1):
    wid = lax.axis_index("s") * NC + lax.axis_index("c")
    base = wid * BPW

    pltpu.sync_copy(lineup_hbm.at[pl.ds(base * L, BPW * L)], lin_v)

    lane = lax.iota(jnp.int32, 16)
    lane21 = lane * L
    idx_b = [idx0_v, idx1_v]
    rv_b = [rv0_v, rv1_v]
    rows_b = [rows0_v, rows1_v]
    sem_b = [sem0, sem1]

    def stage_chunk(c, buf):
        for j in range(S):
            p = plsc.load_gather(lin_v, [lane21 + (c * (CH * L) + j)])
            line = lax.bitwise_or(
                lax.shift_left(lax.shift_right_logical(p, 13), 11),
                lax.bitwise_and(p, 2047),
            )
            off = lax.shift_left(
                lax.bitwise_and(lax.shift_right_logical(p, 11), 3), 5)
            idx_b[buf][j, :] = line
            rv_b[buf][j, :] = off
        for j in range(S):
            pltpu.async_copy(
                table_hbm.at[idx_b[buf].at[j]],
                rows_b[buf].at[pl.ds(j * CH, CH)],
                sem_b[buf],
            )

    def drain_extract(c, buf):
        pltpu.make_async_copy(
            table_hbm.at[pl.ds(0, S * CH)], rows_b[buf], sem_b[buf]
        ).wait()
        tvecs = [lane + (j * CH) for j in range(S)]

        def dim_body(d, carry):
            acc = jnp.zeros((16,), jnp.float32)
            for j in range(S):
                rv = rv_b[buf][j, :]
                acc = acc + plsc.load_gather(rows_b[buf], [tvecs[j], rv + d])
            dvec = jnp.zeros((16,), jnp.int32) + d
            plsc.store_scatter(outc_v, [lane, dvec], acc)
            return carry

        lax.fori_loop(0, D, dim_body, 0)
        pltpu.sync_copy(outc_v, out_hbm.at[pl.ds(base + c * CH, CH)])

    stage_chunk(0, 0)

    def pair_body(g, carry):
        c0 = g * 2
        stage_chunk(c0 + 1, 1)
        drain_extract(c0, 0)

        @pl.when(g < (NCH // 2) - 1)
        def _():
            stage_chunk(c0 + 2, 0)

        drain_extract(c0 + 1, 1)
        return carry

    lax.fori_loop(0, NCH // 2, pair_body, 0)


def _gather_sum(lineup_flat, table_lines):
    mesh = plsc.VectorSubcoreMesh(
        core_axis_name="c", subcore_axis_name="s", num_cores=NC, num_subcores=NS
    )
    return pl.kernel(
        _gather_sum_body,
        out_type=jax.ShapeDtypeStruct((B, D), jnp.float32),
        mesh=mesh,
        scratch_types=[
            pltpu.VMEM((BPW * L,), jnp.int32),
            pltpu.VMEM((S, 16), jnp.int32),
            pltpu.VMEM((S, 16), jnp.int32),
            pltpu.VMEM((S, 16), jnp.int32),
            pltpu.VMEM((S, 16), jnp.int32),
            pltpu.VMEM((S * CH, WIDE), jnp.float32),
            pltpu.VMEM((S * CH, WIDE), jnp.float32),
            pltpu.VMEM((CH, D), jnp.float32),
            pltpu.SemaphoreType.DMA,
            pltpu.SemaphoreType.DMA,
        ],
        compiler_params=pltpu.CompilerParams(needs_layout_passes=False),
    )(lineup_flat, table_lines)


def _mlp_body(x_ref, lineup_ref, w1t_ref, b1_ref, w2_ref, b2_ref, o_ref):
    x = x_ref[...]
    flag = lineup_ref[:, S:].astype(jnp.float32)
    x33 = jnp.concatenate([x, flag], axis=1)
    h = jnp.dot(x33, w1t_ref[...], preferred_element_type=jnp.float32)
    h = jnp.maximum(h + b1_ref[...], 0.0)
    o = jnp.dot(h, w2_ref[...], preferred_element_type=jnp.float32)
    o_ref[...] = o + b2_ref[0, 0]


def _mlp(summed, lineup, w1t, b1_2d, w2t, b2_2d):
    BT = 2048
    grid = (B // BT,)
    return pl.pallas_call(
        _mlp_body,
        grid=grid,
        in_specs=[
            pl.BlockSpec((BT, D), lambda i: (i, 0)),
            pl.BlockSpec((BT, L), lambda i: (i, 0)),
            pl.BlockSpec((D + 1, HIDDEN), lambda i: (0, 0)),
            pl.BlockSpec((1, HIDDEN), lambda i: (0, 0)),
            pl.BlockSpec((HIDDEN, 1), lambda i: (0, 0)),
            pl.BlockSpec((1, 1), lambda i: (0, 0)),
        ],
        out_specs=pl.BlockSpec((BT, 1), lambda i: (i, 0)),
        out_shape=jax.ShapeDtypeStruct((B, 1), jnp.float32),
    )(summed, lineup, w1t, b1_2d, w2t, b2_2d)


def kernel(lineup, table, W1, b1, W2, b2):
    table_lines = _repack(table.T)
    summed = _gather_sum(lineup.reshape(-1), table_lines)
    return _mlp(summed, lineup, W1.T, b1.reshape(1, HIDDEN), W2.T,
                b2.reshape(1, 1))


# speedup vs baseline: 1.5109x; 1.0026x over previous
import jax
import jax.numpy as jnp
from jax import lax
from jax.experimental import pallas as pl
from jax.experimental.pallas import tpu as pltpu
from jax.experimental.pallas import tpu_sc as plsc

B = 16384
L = 21
S = 20
D = 32
HIDDEN = 256
NP = 1000000
WIDE = 128

RBLK = 8192
QB = RBLK // 4
NGRID = (NP + RBLK - 1) // RBLK
NLINES = NGRID * QB

NC, NS = 2, 16
NW = NC * NS
BPW = B // NW
CH = 16
NCH = BPW // CH


def _repack_body(tt_ref, out_ref):
    qs = []
    for q in range(4):
        blk = tt_ref[:, pl.ds(q * QB, QB)]
        qs.append(jnp.transpose(blk, (1, 0)))
    out_ref[...] = jnp.concatenate(qs, axis=1)


def _repack(tableT):
    return pl.pallas_call(
        _repack_body,
        grid=(NGRID,),
        in_specs=[pl.BlockSpec((D, RBLK), lambda i: (0, i))],
        out_specs=pl.BlockSpec((QB, WIDE), lambda i: (i, 0)),
        out_shape=jax.ShapeDtypeStruct((NLINES, WIDE), jnp.float32),
    )(tableT)


def _gather_sum_body(lineup_hbm, table_hbm, out_hbm, lin_v, idx0_v, idx1_v,
                     rv0_v, rv1_v, rows0_v, rows1_v, outc_v, sem0, sem1):
    wid = lax.axis_index("s") * NC + lax.axis_index("c")
    base = wid * BPW

    pltpu.sync_copy(lineup_hbm.at[pl.ds(base * L, BPW * L)], lin_v)

    lane = lax.iota(jnp.int32, 16)
    lane21 = lane * L
    idx_b = [idx0_v, idx1_v]
    rv_b = [rv0_v, rv1_v]
    rows_b = [rows0_v, rows1_v]
    sem_b = [sem0, sem1]

    def stage_chunk(c, buf):
        for j in range(S):
            p = plsc.load_gather(lin_v, [lane21 + (c * (CH * L) + j)])
            line = lax.bitwise_or(
                lax.shift_left(lax.shift_right_logical(p, 13), 11),
                lax.bitwise_and(p, 2047),
            )
            off = lax.shift_left(
                lax.bitwise_and(lax.shift_right_logical(p, 11), 3), 5)
            idx_b[buf][j // 4, pl.ds((j % 4) * 16, 16)] = line
            rv_b[buf][j, :] = off
        for j4 in range(S // 4):
            pltpu.async_copy(
                table_hbm.at[idx_b[buf].at[j4]],
                rows_b[buf].at[pl.ds(j4 * 4 * CH, 4 * CH)],
                sem_b[buf],
            )

    def drain_extract(c, buf):
        pltpu.make_async_copy(
            table_hbm.at[pl.ds(0, S * CH)], rows_b[buf], sem_b[buf]
        ).wait()
        tvecs = [lane + (j * CH) for j in range(S)]

        def dim_body(d, carry):
            acc = jnp.zeros((16,), jnp.float32)
            for j in range(S):
                rv = rv_b[buf][j, :]
                acc = acc + plsc.load_gather(rows_b[buf], [tvecs[j], rv + d])
            dvec = jnp.zeros((16,), jnp.int32) + d
            plsc.store_scatter(outc_v, [lane, dvec], acc)
            return carry

        lax.fori_loop(0, D, dim_body, 0)
        pltpu.sync_copy(outc_v, out_hbm.at[pl.ds(base + c * CH, CH)])

    stage_chunk(0, 0)

    def pair_body(g, carry):
        c0 = g * 2
        stage_chunk(c0 + 1, 1)
        drain_extract(c0, 0)

        @pl.when(g < (NCH // 2) - 1)
        def _():
            stage_chunk(c0 + 2, 0)

        drain_extract(c0 + 1, 1)
        return carry

    lax.fori_loop(0, NCH // 2, pair_body, 0)


def _gather_sum(lineup_flat, table_lines):
    mesh = plsc.VectorSubcoreMesh(
        core_axis_name="c", subcore_axis_name="s", num_cores=NC, num_subcores=NS
    )
    return pl.kernel(
        _gather_sum_body,
        out_type=jax.ShapeDtypeStruct((B, D), jnp.float32),
        mesh=mesh,
        scratch_types=[
            pltpu.VMEM((BPW * L,), jnp.int32),
            pltpu.VMEM((S // 4, 64), jnp.int32),
            pltpu.VMEM((S // 4, 64), jnp.int32),
            pltpu.VMEM((S, 16), jnp.int32),
            pltpu.VMEM((S, 16), jnp.int32),
            pltpu.VMEM((S * CH, WIDE), jnp.float32),
            pltpu.VMEM((S * CH, WIDE), jnp.float32),
            pltpu.VMEM((CH, D), jnp.float32),
            pltpu.SemaphoreType.DMA,
            pltpu.SemaphoreType.DMA,
        ],
        compiler_params=pltpu.CompilerParams(needs_layout_passes=False),
    )(lineup_flat, table_lines)


def _mlp_body(x_ref, lineup_ref, w1t_ref, b1_ref, w2_ref, b2_ref, o_ref):
    x = x_ref[...]
    flag = lineup_ref[:, S:].astype(jnp.float32)
    x33 = jnp.concatenate([x, flag], axis=1)
    h = jnp.dot(x33, w1t_ref[...], preferred_element_type=jnp.float32)
    h = jnp.maximum(h + b1_ref[...], 0.0)
    o = jnp.dot(h, w2_ref[...], preferred_element_type=jnp.float32)
    o_ref[...] = o + b2_ref[0, 0]


def _mlp(summed, lineup, w1t, b1_2d, w2t, b2_2d):
    BT = 2048
    grid = (B // BT,)
    return pl.pallas_call(
        _mlp_body,
        grid=grid,
        in_specs=[
            pl.BlockSpec((BT, D), lambda i: (i, 0)),
            pl.BlockSpec((BT, L), lambda i: (i, 0)),
            pl.BlockSpec((D + 1, HIDDEN), lambda i: (0, 0)),
            pl.BlockSpec((1, HIDDEN), lambda i: (0, 0)),
            pl.BlockSpec((HIDDEN, 1), lambda i: (0, 0)),
            pl.BlockSpec((1, 1), lambda i: (0, 0)),
        ],
        out_specs=pl.BlockSpec((BT, 1), lambda i: (i, 0)),
        out_shape=jax.ShapeDtypeStruct((B, 1), jnp.float32),
    )(summed, lineup, w1t, b1_2d, w2t, b2_2d)


def kernel(lineup, table, W1, b1, W2, b2):
    table_lines = _repack(table.T)
    summed = _gather_sum(lineup.reshape(-1), table_lines)
    return _mlp(summed, lineup, W1.T, b1.reshape(1, HIDDEN), W2.T,
                b2.reshape(1, 1))


# speedup vs baseline: 1.5424x; 1.0209x over previous
import jax
import jax.numpy as jnp
from jax import lax
from jax.experimental import pallas as pl
from jax.experimental.pallas import tpu as pltpu
from jax.experimental.pallas import tpu_sc as plsc

B = 16384
L = 21
S = 20
D = 32
HIDDEN = 256
NP = 1000000
WIDE = 128

RBLK = 8192
QB = RBLK // 4
NGRID = (NP + RBLK - 1) // RBLK
NLINES = NGRID * QB

NC, NS = 2, 16
NW = NC * NS
BPW = B // NW
CH = 16
NCH = BPW // CH


def _repack_body(tt_ref, out_ref):
    qs = []
    for q in range(4):
        blk = tt_ref[:, pl.ds(q * QB, QB)]
        qs.append(jnp.transpose(blk, (1, 0)))
    out_ref[...] = jnp.concatenate(qs, axis=1)


def _repack(tableT):
    return pl.pallas_call(
        _repack_body,
        grid=(NGRID,),
        in_specs=[pl.BlockSpec((D, RBLK), lambda i: (0, i))],
        out_specs=pl.BlockSpec((QB, WIDE), lambda i: (i, 0)),
        out_shape=jax.ShapeDtypeStruct((NLINES, WIDE), jnp.float32),
    )(tableT)


def _gather_sum_body(lineup_hbm, table_hbm, out_hbm, lin_v, idx0_v, idx1_v,
                     rv0_v, rv1_v, rows0_v, rows1_v, outc_v, sem0, sem1):
    wid = lax.axis_index("s") * NC + lax.axis_index("c")
    base = wid * BPW

    pltpu.sync_copy(lineup_hbm.at[pl.ds(base * L, BPW * L)], lin_v)

    lane = lax.iota(jnp.int32, 16)
    lane21 = lane * L
    idx_b = [idx0_v, idx1_v]
    rv_b = [rv0_v, rv1_v]
    rows_b = [rows0_v, rows1_v]
    sem_b = [sem0, sem1]

    def stage_chunk(c, buf):
        for j in range(S):
            p = plsc.load_gather(lin_v, [lane21 + (c * (CH * L) + j)])
            line = lax.bitwise_or(
                lax.shift_left(lax.shift_right_logical(p, 13), 11),
                lax.bitwise_and(p, 2047),
            )
            off = lax.shift_left(
                lax.bitwise_and(lax.shift_right_logical(p, 11), 3), 5)
            idx_b[buf][j // 4, pl.ds((j % 4) * 16, 16)] = line
            rv_b[buf][j, :] = off
        for j4 in range(S // 4):
            pltpu.async_copy(
                table_hbm.at[idx_b[buf].at[j4]],
                rows_b[buf].at[pl.ds(j4 * 4 * CH, 4 * CH)],
                sem_b[buf],
            )

    def drain_extract(c, buf):
        pltpu.make_async_copy(
            table_hbm.at[pl.ds(0, S * CH)], rows_b[buf], sem_b[buf]
        ).wait()
        zero16 = jnp.zeros((16,), jnp.int32)
        fvecs = [(lane + (j * CH)) * WIDE + rv_b[buf][j, :] for j in range(S)]

        def dim_body(d, carry):
            acc = jnp.zeros((16,), jnp.float32)
            for j in range(S):
                acc = acc + plsc.load_gather(rows_b[buf], [zero16, fvecs[j] + d])
            plsc.store_scatter(outc_v, [lane, zero16 + d], acc)
            return carry

        lax.fori_loop(0, D, dim_body, 0)
        pltpu.sync_copy(outc_v, out_hbm.at[pl.ds(base + c * CH, CH)])

    stage_chunk(0, 0)

    def pair_body(g, carry):
        c0 = g * 2
        stage_chunk(c0 + 1, 1)
        drain_extract(c0, 0)

        @pl.when(g < (NCH // 2) - 1)
        def _():
            stage_chunk(c0 + 2, 0)

        drain_extract(c0 + 1, 1)
        return carry

    lax.fori_loop(0, NCH // 2, pair_body, 0)


def _gather_sum(lineup_flat, table_lines):
    mesh = plsc.VectorSubcoreMesh(
        core_axis_name="c", subcore_axis_name="s", num_cores=NC, num_subcores=NS
    )
    return pl.kernel(
        _gather_sum_body,
        out_type=jax.ShapeDtypeStruct((B, D), jnp.float32),
        mesh=mesh,
        scratch_types=[
            pltpu.VMEM((BPW * L,), jnp.int32),
            pltpu.VMEM((S // 4, 64), jnp.int32),
            pltpu.VMEM((S // 4, 64), jnp.int32),
            pltpu.VMEM((S, 16), jnp.int32),
            pltpu.VMEM((S, 16), jnp.int32),
            pltpu.VMEM((S * CH, WIDE), jnp.float32),
            pltpu.VMEM((S * CH, WIDE), jnp.float32),
            pltpu.VMEM((CH, D), jnp.float32),
            pltpu.SemaphoreType.DMA,
            pltpu.SemaphoreType.DMA,
        ],
        compiler_params=pltpu.CompilerParams(needs_layout_passes=False),
    )(lineup_flat, table_lines)


def _mlp_body(x_ref, lineup_ref, w1t_ref, b1_ref, w2_ref, b2_ref, o_ref):
    x = x_ref[...]
    flag = lineup_ref[:, S:].astype(jnp.float32)
    x33 = jnp.concatenate([x, flag], axis=1)
    h = jnp.dot(x33, w1t_ref[...], preferred_element_type=jnp.float32)
    h = jnp.maximum(h + b1_ref[...], 0.0)
    o = jnp.dot(h, w2_ref[...], preferred_element_type=jnp.float32)
    o_ref[...] = o + b2_ref[0, 0]


def _mlp(summed, lineup, w1t, b1_2d, w2t, b2_2d):
    BT = 2048
    grid = (B // BT,)
    return pl.pallas_call(
        _mlp_body,
        grid=grid,
        in_specs=[
            pl.BlockSpec((BT, D), lambda i: (i, 0)),
            pl.BlockSpec((BT, L), lambda i: (i, 0)),
            pl.BlockSpec((D + 1, HIDDEN), lambda i: (0, 0)),
            pl.BlockSpec((1, HIDDEN), lambda i: (0, 0)),
            pl.BlockSpec((HIDDEN, 1), lambda i: (0, 0)),
            pl.BlockSpec((1, 1), lambda i: (0, 0)),
        ],
        out_specs=pl.BlockSpec((BT, 1), lambda i: (i, 0)),
        out_shape=jax.ShapeDtypeStruct((B, 1), jnp.float32),
    )(summed, lineup, w1t, b1_2d, w2t, b2_2d)


def kernel(lineup, table, W1, b1, W2, b2):
    table_lines = _repack(table.T)
    summed = _gather_sum(lineup.reshape(-1), table_lines)
    return _mlp(summed, lineup, W1.T, b1.reshape(1, HIDDEN), W2.T,
                b2.reshape(1, 1))


# speedup vs baseline: 1.5586x; 1.0105x over previous
import jax
import jax.numpy as jnp
from jax import lax
from jax.experimental import pallas as pl
from jax.experimental.pallas import tpu as pltpu
from jax.experimental.pallas import tpu_sc as plsc

B = 16384
L = 21
S = 20
D = 32
HIDDEN = 256
NP = 1000000
WIDE = 128

RBLK = 32768
RB_LOG = 15
QB = RBLK // 4
QB_LOG = RB_LOG - 2
NGRID = (NP + RBLK - 1) // RBLK
NLINES = NGRID * QB

NC, NS = 2, 16
NW = NC * NS
BPW = B // NW
CH = 16
NCH = BPW // CH


def _repack_body(tt_ref, out_ref):
    qs = []
    for q in range(4):
        blk = tt_ref[:, pl.ds(q * QB, QB)]
        qs.append(jnp.transpose(blk, (1, 0)))
    out_ref[...] = jnp.concatenate(qs, axis=1)


def _repack(tableT):
    return pl.pallas_call(
        _repack_body,
        grid=(NGRID,),
        in_specs=[pl.BlockSpec((D, RBLK), lambda i: (0, i))],
        out_specs=pl.BlockSpec((QB, WIDE), lambda i: (i, 0)),
        out_shape=jax.ShapeDtypeStruct((NLINES, WIDE), jnp.float32),
    )(tableT)


def _gather_sum_body(lineup_hbm, table_hbm, out_hbm, lin_v, idx0_v, idx1_v,
                     rv0_v, rv1_v, rows0_v, rows1_v, outc_v, sem0, sem1):
    wid = lax.axis_index("s") * NC + lax.axis_index("c")
    base = wid * BPW

    pltpu.sync_copy(lineup_hbm.at[pl.ds(base * L, BPW * L)], lin_v)

    lane = lax.iota(jnp.int32, 16)
    lane21 = lane * L
    idx_b = [idx0_v, idx1_v]
    rv_b = [rv0_v, rv1_v]
    rows_b = [rows0_v, rows1_v]
    sem_b = [sem0, sem1]

    def stage_chunk(c, buf):
        for j in range(S):
            p = plsc.load_gather(lin_v, [lane21 + (c * (CH * L) + j)])
            line = lax.bitwise_or(
                lax.shift_left(lax.shift_right_logical(p, RB_LOG), QB_LOG),
                lax.bitwise_and(p, QB - 1),
            )
            off = lax.shift_left(
                lax.bitwise_and(lax.shift_right_logical(p, QB_LOG), 3), 5)
            idx_b[buf][j // 4, pl.ds((j % 4) * 16, 16)] = line
            rv_b[buf][j, :] = off
        for j4 in range(S // 4):
            pltpu.async_copy(
                table_hbm.at[idx_b[buf].at[j4]],
                rows_b[buf].at[pl.ds(j4 * 4 * CH, 4 * CH)],
                sem_b[buf],
            )

    def drain_extract(c, buf):
        pltpu.make_async_copy(
            table_hbm.at[pl.ds(0, S * CH)], rows_b[buf], sem_b[buf]
        ).wait()
        zero16 = jnp.zeros((16,), jnp.int32)
        fvecs = [(lane + (j * CH)) * WIDE + rv_b[buf][j, :] for j in range(S)]

        def dim_body(d, carry):
            acc = jnp.zeros((16,), jnp.float32)
            for j in range(S):
                acc = acc + plsc.load_gather(rows_b[buf], [zero16, fvecs[j] + d])
            plsc.store_scatter(outc_v, [lane, zero16 + d], acc)
            return carry

        lax.fori_loop(0, D, dim_body, 0)
        pltpu.sync_copy(outc_v, out_hbm.at[pl.ds(base + c * CH, CH)])

    stage_chunk(0, 0)

    def pair_body(g, carry):
        c0 = g * 2
        stage_chunk(c0 + 1, 1)
        drain_extract(c0, 0)

        @pl.when(g < (NCH // 2) - 1)
        def _():
            stage_chunk(c0 + 2, 0)

        drain_extract(c0 + 1, 1)
        return carry

    lax.fori_loop(0, NCH // 2, pair_body, 0)


def _gather_sum(lineup_flat, table_lines):
    mesh = plsc.VectorSubcoreMesh(
        core_axis_name="c", subcore_axis_name="s", num_cores=NC, num_subcores=NS
    )
    return pl.kernel(
        _gather_sum_body,
        out_type=jax.ShapeDtypeStruct((B, D), jnp.float32),
        mesh=mesh,
        scratch_types=[
            pltpu.VMEM((BPW * L,), jnp.int32),
            pltpu.VMEM((S // 4, 64), jnp.int32),
            pltpu.VMEM((S // 4, 64), jnp.int32),
            pltpu.VMEM((S, 16), jnp.int32),
            pltpu.VMEM((S, 16), jnp.int32),
            pltpu.VMEM((S * CH, WIDE), jnp.float32),
            pltpu.VMEM((S * CH, WIDE), jnp.float32),
            pltpu.VMEM((CH, D), jnp.float32),
            pltpu.SemaphoreType.DMA,
            pltpu.SemaphoreType.DMA,
        ],
        compiler_params=pltpu.CompilerParams(needs_layout_passes=False),
    )(lineup_flat, table_lines)


def _mlp_body(x_ref, lineup_ref, w1t_ref, b1_ref, w2_ref, b2_ref, o_ref):
    x = x_ref[...]
    flag = lineup_ref[:, S:].astype(jnp.float32)
    x33 = jnp.concatenate([x, flag], axis=1)
    h = jnp.dot(x33, w1t_ref[...], preferred_element_type=jnp.float32)
    h = jnp.maximum(h + b1_ref[...], 0.0)
    o = jnp.dot(h, w2_ref[...], preferred_element_type=jnp.float32)
    o_ref[...] = o + b2_ref[0, 0]


def _mlp(summed, lineup, w1t, b1_2d, w2t, b2_2d):
    BT = 2048
    grid = (B // BT,)
    return pl.pallas_call(
        _mlp_body,
        grid=grid,
        in_specs=[
            pl.BlockSpec((BT, D), lambda i: (i, 0)),
            pl.BlockSpec((BT, L), lambda i: (i, 0)),
            pl.BlockSpec((D + 1, HIDDEN), lambda i: (0, 0)),
            pl.BlockSpec((1, HIDDEN), lambda i: (0, 0)),
            pl.BlockSpec((HIDDEN, 1), lambda i: (0, 0)),
            pl.BlockSpec((1, 1), lambda i: (0, 0)),
        ],
        out_specs=pl.BlockSpec((BT, 1), lambda i: (i, 0)),
        out_shape=jax.ShapeDtypeStruct((B, 1), jnp.float32),
    )(summed, lineup, w1t, b1_2d, w2t, b2_2d)


def kernel(lineup, table, W1, b1, W2, b2):
    table_lines = _repack(table.T)
    summed = _gather_sum(lineup.reshape(-1), table_lines)
    return _mlp(summed, lineup, W1.T, b1.reshape(1, HIDDEN), W2.T,
                b2.reshape(1, 1))
